# baseline (speedup 1.0000x reference)
import functools

import jax
import jax.numpy as jnp
from jax import lax
from jax.experimental import pallas as pl
from jax.experimental.pallas import tpu as pltpu
from jax.experimental.pallas import tpu_sc as plsc

N = 10000
K = 32
D = 128
NW = 32
NPAD = 10240
BPW = NPAD // NW
CH = 4
NCH = BPW // CH
NST = NCH // 2
NV = D // 16


def _sc_agg(x_pad, idx3):
    mesh = plsc.VectorSubcoreMesh(core_axis_name="c", subcore_axis_name="s")

    @functools.partial(
        pl.kernel,
        mesh=mesh,
        out_type=jax.ShapeDtypeStruct((NW, BPW, D), jnp.float32),
        scratch_types=[
            pltpu.VMEM((NCH, CH * K), jnp.int32),
            pltpu.VMEM((CH * K, D), jnp.float32),
            pltpu.VMEM((CH * K, D), jnp.float32),
            pltpu.VMEM((2 * CH, D), jnp.float32),
            pltpu.VMEM((2 * CH, D), jnp.float32),
            pltpu.VMEM_SHARED((N, D), jnp.float32),
            pltpu.SemaphoreType.DMA,
            pltpu.SemaphoreType.DMA,
            pltpu.SemaphoreType.DMA,
            pltpu.SemaphoreType.DMA,
        ],
    )
    def body(x_hbm, idx_hbm, out_hbm, idx_v, buf0, buf1, oc0, oc1,
             x_sh, sem0, sem1, semo0, semo1):
        sid = lax.axis_index("s")
        wid = sid * 2 + lax.axis_index("c")
        rows = N // 16 - 1
        pltpu.sync_copy(
            x_hbm.at[pl.ds(sid * rows, rows)], x_sh.at[pl.ds(sid * rows, rows)]
        )

        @pl.when(sid == 15)
        def _():
            pltpu.sync_copy(
                x_hbm.at[pl.ds(16 * rows, N - 16 * rows)],
                x_sh.at[pl.ds(16 * rows, N - 16 * rows)],
            )
        pltpu.sync_copy(idx_hbm.at[wid], idx_v)
        plsc.subcore_barrier()

        zero = jnp.zeros((16,), jnp.float32)

        def process(buf, oc, half):
            for i in range(CH):
                row0 = i * K
                accs = (zero,) * NV

                def rbody(q, a):
                    a = list(a)
                    for u in range(16):
                        row = row0 + 16 * q + u
                        for d in range(NV):
                            a[d] = a[d] + buf[row, pl.ds(d * 16, 16)]
                    return tuple(a)

                accs = lax.fori_loop(0, K // 16, rbody, accs)
                for d in range(NV):
                    oc[half * CH + i, pl.ds(d * 16, 16)] = accs[d]

        pltpu.async_copy(x_sh.at[idx_v.at[0]], buf0, sem0)

        def step(s, carry):
            c0 = s * 2
            even = s % 2 == 0

            def run(ocbuf, osem):
                @pl.when(s >= 2)
                def _():
                    pltpu.make_async_copy(
                        ocbuf, out_hbm.at[wid, pl.ds(0, 2 * CH)], osem
                    ).wait()

                pltpu.make_async_copy(x_sh.at[idx_v.at[c0]], buf0, sem0).wait()
                pltpu.async_copy(x_sh.at[idx_v.at[c0 + 1]], buf1, sem1)
                process(buf0, ocbuf, 0)
                pltpu.make_async_copy(
                    x_sh.at[idx_v.at[c0 + 1]], buf1, sem1
                ).wait()

                @pl.when(s < NST - 1)
                def _():
                    pltpu.async_copy(x_sh.at[idx_v.at[c0 + 2]], buf0, sem0)

                process(buf1, ocbuf, 1)
                pltpu.async_copy(
                    ocbuf, out_hbm.at[wid, pl.ds(c0 * CH, 2 * CH)], osem
                )

            @pl.when(even)
            def _():
                run(oc0, semo0)

            @pl.when(jnp.logical_not(even))
            def _():
                run(oc1, semo1)

            return carry

        lax.fori_loop(0, NST, step, 0)
        pltpu.make_async_copy(
            oc0, out_hbm.at[wid, pl.ds(0, 2 * CH)], semo0
        ).wait()
        pltpu.make_async_copy(
            oc1, out_hbm.at[wid, pl.ds(0, 2 * CH)], semo1
        ).wait()

    return body(x_pad, idx3)


def _tc_out(x, aggsum, weight):
    R = 1024

    def body(x_ref, a_ref, w_ref, o_ref):
        w1 = w_ref[:D, :].astype(jnp.bfloat16)
        w2 = (w_ref[D:, :] * (1.0 / K)).astype(jnp.bfloat16)
        o_ref[...] = (
            jnp.dot(x_ref[...].astype(jnp.bfloat16), w1,
                    preferred_element_type=jnp.float32)
            + jnp.dot(a_ref[...].astype(jnp.bfloat16), w2,
                      preferred_element_type=jnp.float32)
        )

    return pl.pallas_call(
        body,
        grid=(NPAD // R,),
        in_specs=[
            pl.BlockSpec((R, D), lambda i: (i, 0)),
            pl.BlockSpec((R, D), lambda i: (i, 0)),
            pl.BlockSpec((2 * D, D), lambda i: (0, 0)),
        ],
        out_specs=pl.BlockSpec((R, D), lambda i: (i, 0)),
        out_shape=jax.ShapeDtypeStruct((N, D), jnp.float32),
    )(x, aggsum, weight)


def kernel(x, adj_list, weight):
    adj = adj_list.astype(jnp.int32).T
    adj = jnp.pad(adj, ((0, NPAD - N), (0, 0)))
    idx3 = adj.reshape(NW, NCH, CH * K)
    aggsum = _sc_agg(x, idx3).reshape(NPAD, D)
    return _tc_out(x, aggsum, weight)


# speedup vs baseline: 1.1322x; 1.1322x over previous
import functools

import jax
import jax.numpy as jnp
from jax import lax
from jax.experimental import pallas as pl
from jax.experimental.pallas import tpu as pltpu
from jax.experimental.pallas import tpu_sc as plsc

N = 10000
K = 32
D = 128
NW = 32
NPAD = 10240
BPW = NPAD // NW
CH = 4
NCH = BPW // CH
NST = NCH // 2
NV = D // 16


def _sc_agg(x_pad, idx3):
    mesh = plsc.VectorSubcoreMesh(core_axis_name="c", subcore_axis_name="s")

    @functools.partial(
        pl.kernel,
        mesh=mesh,
        out_type=jax.ShapeDtypeStruct((NW, BPW, D), jnp.float32),
        scratch_types=[
            pltpu.VMEM((NCH, CH * K), jnp.int32),
            pltpu.VMEM((CH * K, D), jnp.float32),
            pltpu.VMEM((CH * K, D), jnp.float32),
            pltpu.VMEM((2 * CH, D), jnp.float32),
            pltpu.VMEM((2 * CH, D), jnp.float32),
            pltpu.VMEM_SHARED((N, D), jnp.float32),
            pltpu.SemaphoreType.DMA,
            pltpu.SemaphoreType.DMA,
            pltpu.SemaphoreType.DMA,
            pltpu.SemaphoreType.DMA,
        ],
    )
    def body(x_hbm, idx_hbm, out_hbm, idx_v, buf0, buf1, oc0, oc1,
             x_sh, sem0, sem1, semo0, semo1):
        sid = lax.axis_index("s")
        wid = sid * 2 + lax.axis_index("c")
        rows = N // 16 - 1
        pltpu.sync_copy(
            x_hbm.at[pl.ds(sid * rows, rows)], x_sh.at[pl.ds(sid * rows, rows)]
        )

        @pl.when(sid == 15)
        def _():
            pltpu.sync_copy(
                x_hbm.at[pl.ds(16 * rows, N - 16 * rows)],
                x_sh.at[pl.ds(16 * rows, N - 16 * rows)],
            )
        pltpu.sync_copy(idx_hbm.at[wid], idx_v)
        plsc.subcore_barrier()

        zero = jnp.zeros((16,), jnp.float32)

        def process(buf, oc, half):
            for i in range(CH):
                row0 = i * K
                accs = (zero,) * NV

                def rbody(q, a):
                    a = list(a)
                    for u in range(8):
                        row = row0 + 8 * q + u
                        for d in range(NV):
                            a[d] = a[d] + buf[row, pl.ds(d * 16, 16)]
                    return tuple(a)

                accs = lax.fori_loop(0, K // 8, rbody, accs)
                for d in range(NV):
                    oc[half * CH + i, pl.ds(d * 16, 16)] = accs[d]

        pltpu.async_copy(x_sh.at[idx_v.at[0]], buf0, sem0)

        def step(s, carry):
            c0 = s * 2
            even = s % 2 == 0

            def run(ocbuf, osem):
                @pl.when(s >= 2)
                def _():
                    pltpu.make_async_copy(
                        ocbuf, out_hbm.at[wid, pl.ds(0, 2 * CH)], osem
                    ).wait()

                pltpu.make_async_copy(x_sh.at[idx_v.at[c0]], buf0, sem0).wait()
                pltpu.async_copy(x_sh.at[idx_v.at[c0 + 1]], buf1, sem1)
                process(buf0, ocbuf, 0)
                pltpu.make_async_copy(
                    x_sh.at[idx_v.at[c0 + 1]], buf1, sem1
                ).wait()

                @pl.when(s < NST - 1)
                def _():
                    pltpu.async_copy(x_sh.at[idx_v.at[c0 + 2]], buf0, sem0)

                process(buf1, ocbuf, 1)
                pltpu.async_copy(
                    ocbuf, out_hbm.at[wid, pl.ds(c0 * CH, 2 * CH)], osem
                )

            @pl.when(even)
            def _():
                run(oc0, semo0)

            @pl.when(jnp.logical_not(even))
            def _():
                run(oc1, semo1)

            return carry

        lax.fori_loop(0, NST, step, 0)
        pltpu.make_async_copy(
            oc0, out_hbm.at[wid, pl.ds(0, 2 * CH)], semo0
        ).wait()
        pltpu.make_async_copy(
            oc1, out_hbm.at[wid, pl.ds(0, 2 * CH)], semo1
        ).wait()

    return body(x_pad, idx3)


def _tc_out(x, aggsum, weight):
    R = 1024

    def body(x_ref, a_ref, w_ref, o_ref):
        w1 = w_ref[:D, :]
        w2 = w_ref[D:, :] * (1.0 / K)
        o_ref[...] = (
            jnp.dot(x_ref[...], w1, preferred_element_type=jnp.float32)
            + jnp.dot(a_ref[...], w2, preferred_element_type=jnp.float32)
        )

    return pl.pallas_call(
        body,
        grid=(NPAD // R,),
        in_specs=[
            pl.BlockSpec((R, D), lambda i: (i, 0)),
            pl.BlockSpec((R, D), lambda i: (i, 0)),
            pl.BlockSpec((2 * D, D), lambda i: (0, 0)),
        ],
        out_specs=pl.BlockSpec((R, D), lambda i: (i, 0)),
        out_shape=jax.ShapeDtypeStruct((N, D), jnp.float32),
    )(x, aggsum, weight)


def kernel(x, adj_list, weight):
    adj = adj_list.astype(jnp.int32).T
    adj = jnp.pad(adj, ((0, NPAD - N), (0, 0)))
    idx3 = adj.reshape(NW, NCH, CH * K)
    aggsum = _sc_agg(x, idx3).reshape(NPAD, D)
    return _tc_out(x, aggsum, weight)
